# Initial kernel scaffold; baseline (speedup 1.0000x reference)
import jax
import jax.numpy as jnp
from jax.experimental import pallas as pl


def kernel(grid, value_embed, pos_encoding):
    raise NotImplementedError("write your pallas kernel here")


# speedup vs baseline: 6.4724x; 6.4724x over previous
import jax
import jax.numpy as jnp
from jax.experimental import pallas as pl
from jax.experimental.pallas import tpu as pltpu

B, H, W = 1024, 30, 30
NV, DV = 10, 64
DP = 64
D = 128
BB = 8


def _embed_body(g_ref, ve_ref, pos_ref, out_ref):
    g = g_ref[...][..., None]
    val = jnp.broadcast_to(ve_ref[0][None, None, None, :], (BB, H, W, DV))
    for v in range(1, NV):
        row = jnp.broadcast_to(ve_ref[v][None, None, None, :], (BB, H, W, DV))
        val = jnp.where(g == v, row, val)
    pos = jnp.broadcast_to(pos_ref[...][None], (BB, H, W, DP))
    out_ref[...] = jnp.concatenate([val, pos], axis=-1)


def kernel(grid, value_embed, pos_encoding):
    g32 = grid.astype(jnp.int32)
    out = pl.pallas_call(
        _embed_body,
        grid=(B // BB,),
        in_specs=[
            pl.BlockSpec((BB, H, W), lambda i: (i, 0, 0)),
            pl.BlockSpec((NV, DV), lambda i: (0, 0)),
            pl.BlockSpec((H, W, DP), lambda i: (0, 0, 0)),
        ],
        out_specs=pl.BlockSpec((BB, H, W, D), lambda i: (i, 0, 0, 0)),
        out_shape=jax.ShapeDtypeStruct((B, H, W, D), jnp.float32),
    )(g32, value_embed, pos_encoding)
    return out
